# Initial kernel scaffold; baseline (speedup 1.0000x reference)
import jax
import jax.numpy as jnp
from jax.experimental import pallas as pl


def kernel(words, emb_table, W_fc, b_fc):
    raise NotImplementedError("write your pallas kernel here")


# speedup vs baseline: 1.7894x; 1.7894x over previous
import functools

import jax
import jax.numpy as jnp
from jax import lax
from jax.experimental import pallas as pl
from jax.experimental.pallas import tpu as pltpu
from jax.experimental.pallas import tpu_sc as plsc

VOCAB = 1000000
NCLASS = 1000
EMBED = 32
CTX = 50
BATCH = 16384

NC = 2
NS = 16
NW = NC * NS
BPW = BATCH // NW
BC = 32
NCH = BPW // BC
GW = 100
GPC = BC * CTX // GW
HALF = EMBED // 2


def _sc_embed_sum_body(words_hbm, emb_hbm, out_hbm, idx_v, rows_v, out_v, sem_g):
    w = lax.axis_index("s") * NC + lax.axis_index("c")

    def chunk_body(ch, _):
        pltpu.sync_copy(words_hbm.at[w, ch], idx_v)
        copies = []
        for g in range(GPC):
            copies.append(pltpu.async_copy(
                emb_hbm.at[idx_v.at[g]],
                rows_v.at[pl.ds(g * GW, GW)],
                sem_g,
            ))
        for cp in copies:
            cp.wait()

        def elem_body(e, _):
            r0 = e * CTX
            a0 = rows_v[r0, pl.ds(0, HALF)]
            a1 = rows_v[r0, pl.ds(HALF, HALF)]
            b0 = rows_v[r0 + 1, pl.ds(0, HALF)]
            b1 = rows_v[r0 + 1, pl.ds(HALF, HALF)]
            for c in range(2, CTX, 2):
                a0 = a0 + rows_v[r0 + c, pl.ds(0, HALF)]
                a1 = a1 + rows_v[r0 + c, pl.ds(HALF, HALF)]
                b0 = b0 + rows_v[r0 + c + 1, pl.ds(0, HALF)]
                b1 = b1 + rows_v[r0 + c + 1, pl.ds(HALF, HALF)]
            out_v[e, pl.ds(0, HALF)] = a0 + b0
            out_v[e, pl.ds(HALF, HALF)] = a1 + b1
            return _

        lax.fori_loop(0, BC, elem_body, 0, unroll=False)
        pltpu.sync_copy(out_v, out_hbm.at[pl.ds(w * BPW + ch * BC, BC)])
        return _

    lax.fori_loop(0, NCH, chunk_body, 0, unroll=False)


@jax.jit
def _sc_embed_sum(words_grouped, emb_table):
    return pl.kernel(
        _sc_embed_sum_body,
        out_type=jax.ShapeDtypeStruct((BATCH, EMBED), jnp.float32),
        mesh=plsc.VectorSubcoreMesh(
            core_axis_name="c", subcore_axis_name="s",
            num_cores=NC, num_subcores=NS),
        scratch_types=[
            pltpu.VMEM((GPC, GW), jnp.int32),
            pltpu.VMEM((BC * CTX, EMBED), jnp.float32),
            pltpu.VMEM((BC, EMBED), jnp.float32),
            pltpu.SemaphoreType.DMA,
        ],
        compiler_params=pltpu.CompilerParams(use_tc_tiling_on_sc=False),
    )(words_grouped, emb_table)


def _tc_fc_body(x_ref, w_ref, b_ref, o_ref):
    o_ref[...] = jnp.dot(
        x_ref[...], w_ref[...], preferred_element_type=jnp.float32
    ) + b_ref[...]


@jax.jit
def _tc_fc(embed_sum, w_t, b_row):
    bm = 2048
    return pl.pallas_call(
        _tc_fc_body,
        grid=(BATCH // bm,),
        in_specs=[
            pl.BlockSpec((bm, EMBED), lambda i: (i, 0)),
            pl.BlockSpec((EMBED, NCLASS), lambda i: (0, 0)),
            pl.BlockSpec((1, NCLASS), lambda i: (0, 0)),
        ],
        out_specs=pl.BlockSpec((bm, NCLASS), lambda i: (i, 0)),
        out_shape=jax.ShapeDtypeStruct((BATCH, NCLASS), jnp.float32),
    )(embed_sum, w_t, b_row)


def kernel(words, emb_table, W_fc, b_fc):
    words_grouped = words.astype(jnp.int32).T.reshape(NW, NCH, GPC, GW)
    embed_sum = _sc_embed_sum(words_grouped, emb_table)
    return _tc_fc(embed_sum, W_fc.T, b_fc.reshape(1, NCLASS))


# speedup vs baseline: 1.9526x; 1.0912x over previous
import functools

import jax
import jax.numpy as jnp
from jax import lax
from jax.experimental import pallas as pl
from jax.experimental.pallas import tpu as pltpu
from jax.experimental.pallas import tpu_sc as plsc

VOCAB = 1000000
NCLASS = 1000
EMBED = 32
CTX = 50
BATCH = 16384

NC = 2
NS = 16
NW = NC * NS
BPW = BATCH // NW
BC = 32
NCH = BPW // BC
GW = 100
GPC = BC * CTX // GW
HALF = EMBED // 2


def _sc_embed_sum_body(words_hbm, emb_hbm, out_hbm, idx_v, rows_v, out_v, sem_g):
    w = lax.axis_index("s") * NC + lax.axis_index("c")

    def chunk_body(ch, _):
        pltpu.sync_copy(words_hbm.at[w, ch], idx_v)
        copies = []
        for g in range(GPC):
            copies.append(pltpu.async_copy(
                emb_hbm.at[idx_v.at[g]],
                rows_v.at[pl.ds(g * GW, GW)],
                sem_g,
            ))
        for cp in copies:
            cp.wait()

        def elem_body(e, _):
            r0 = e * CTX
            a0 = rows_v[r0, pl.ds(0, HALF)]
            a1 = rows_v[r0, pl.ds(HALF, HALF)]
            b0 = rows_v[r0 + 1, pl.ds(0, HALF)]
            b1 = rows_v[r0 + 1, pl.ds(HALF, HALF)]
            for c in range(2, CTX, 2):
                a0 = a0 + rows_v[r0 + c, pl.ds(0, HALF)]
                a1 = a1 + rows_v[r0 + c, pl.ds(HALF, HALF)]
                b0 = b0 + rows_v[r0 + c + 1, pl.ds(0, HALF)]
                b1 = b1 + rows_v[r0 + c + 1, pl.ds(HALF, HALF)]
            out_v[e, pl.ds(0, HALF)] = a0 + b0
            out_v[e, pl.ds(HALF, HALF)] = a1 + b1
            return _

        lax.fori_loop(0, BC, elem_body, 0, unroll=False)
        pltpu.sync_copy(out_v, out_hbm.at[pl.ds(w * BPW + ch * BC, BC)])
        return _

    lax.fori_loop(0, NCH, chunk_body, 0, unroll=False)


@jax.jit
def _sc_embed_sum(words_grouped, emb_table):
    return pl.kernel(
        _sc_embed_sum_body,
        out_type=jax.ShapeDtypeStruct((BATCH, EMBED), jnp.float32),
        mesh=plsc.VectorSubcoreMesh(
            core_axis_name="c", subcore_axis_name="s",
            num_cores=NC, num_subcores=NS),
        scratch_types=[
            pltpu.VMEM((GPC, GW), jnp.int32),
            pltpu.VMEM((BC * CTX, EMBED), jnp.float32),
            pltpu.VMEM((BC, EMBED), jnp.float32),
            pltpu.SemaphoreType.DMA,
        ],
        compiler_params=pltpu.CompilerParams(use_tc_tiling_on_sc=False),
    )(words_grouped, emb_table)


def _tc_fc_body(w_ref, x_ref, b_ref, o_ref):
    o_ref[...] = lax.dot_general(
        w_ref[...], x_ref[...],
        (((1,), (1,)), ((), ())),
        preferred_element_type=jnp.float32,
    ) + b_ref[...]


@jax.jit
def _tc_fc(embed_sum, w_fc, b_col):
    bm = 2048
    out_t = pl.pallas_call(
        _tc_fc_body,
        grid=(BATCH // bm,),
        in_specs=[
            pl.BlockSpec((NCLASS, EMBED), lambda i: (0, 0)),
            pl.BlockSpec((bm, EMBED), lambda i: (i, 0)),
            pl.BlockSpec((NCLASS, 1), lambda i: (0, 0)),
        ],
        out_specs=pl.BlockSpec((NCLASS, bm), lambda i: (0, i)),
        out_shape=jax.ShapeDtypeStruct((NCLASS, BATCH), jnp.float32),
    )(w_fc, embed_sum, b_col)
    return out_t.T


def kernel(words, emb_table, W_fc, b_fc):
    words_grouped = words.astype(jnp.int32).T.reshape(NW, NCH, GPC, GW)
    embed_sum = _sc_embed_sum(words_grouped, emb_table)
    return _tc_fc(embed_sum, W_fc, b_fc.reshape(NCLASS, 1))


# speedup vs baseline: 3.0343x; 1.5540x over previous
import functools

import jax
import jax.numpy as jnp
from jax import lax
from jax.experimental import pallas as pl
from jax.experimental.pallas import tpu as pltpu
from jax.experimental.pallas import tpu_sc as plsc

VOCAB = 1000000
NCLASS = 1000
EMBED = 32
CTX = 50
BATCH = 16384

NC = 2
NS = 16
NW = NC * NS
BPW = BATCH // NW
BC = 32
NCH = BPW // BC
GW = 100
GPC = BC * CTX // GW
HALF = EMBED // 2


def _sc_embed_sum_body(words_hbm, emb_hbm, out_hbm, idx_v, rows_v, out_v, sem_g):
    w = lax.axis_index("s") * NC + lax.axis_index("c")

    def chunk_body(ch, _):
        pltpu.sync_copy(words_hbm.at[w, ch], idx_v)
        copies = []
        for g in range(GPC):
            copies.append(pltpu.async_copy(
                emb_hbm.at[idx_v.at[g]],
                rows_v.at[pl.ds(g * GW, GW)],
                sem_g,
            ))
        for cp in copies:
            cp.wait()

        def elem_body(e, _):
            r0 = e * CTX
            a0 = rows_v[r0, pl.ds(0, HALF)]
            a1 = rows_v[r0, pl.ds(HALF, HALF)]
            b0 = rows_v[r0 + 1, pl.ds(0, HALF)]
            b1 = rows_v[r0 + 1, pl.ds(HALF, HALF)]
            for c in range(2, CTX, 2):
                a0 = a0 + rows_v[r0 + c, pl.ds(0, HALF)]
                a1 = a1 + rows_v[r0 + c, pl.ds(HALF, HALF)]
                b0 = b0 + rows_v[r0 + c + 1, pl.ds(0, HALF)]
                b1 = b1 + rows_v[r0 + c + 1, pl.ds(HALF, HALF)]
            out_v[e, pl.ds(0, HALF)] = a0 + b0
            out_v[e, pl.ds(HALF, HALF)] = a1 + b1
            return _

        lax.fori_loop(0, BC, elem_body, 0, unroll=False)
        pltpu.sync_copy(out_v, out_hbm.at[pl.ds(w * BPW + ch * BC, BC)])
        return _

    lax.fori_loop(0, NCH, chunk_body, 0, unroll=False)


@jax.jit
def _sc_embed_sum(words_grouped, emb_table):
    return pl.kernel(
        _sc_embed_sum_body,
        out_type=jax.ShapeDtypeStruct((BATCH, EMBED), jnp.float32),
        mesh=plsc.VectorSubcoreMesh(
            core_axis_name="c", subcore_axis_name="s",
            num_cores=NC, num_subcores=NS),
        scratch_types=[
            pltpu.VMEM((GPC, GW), jnp.int32),
            pltpu.VMEM((BC * CTX, EMBED), jnp.float32),
            pltpu.VMEM((BC, EMBED), jnp.float32),
            pltpu.SemaphoreType.DMA,
        ],
        compiler_params=pltpu.CompilerParams(use_tc_tiling_on_sc=False),
    )(words_grouped, emb_table)


VB = 16384
RB = VB // 4
NTB = (VOCAB + VB - 1) // VB
VOCAB2 = NTB * VB


def _tt_body(x_ref, o_ref):
    xt = x_ref[...].T
    o_ref[...] = jnp.concatenate(
        [xt[j * RB:(j + 1) * RB, :] for j in range(4)], axis=1)


@jax.jit
def _tc_table_transpose(emb_t):
    return pl.pallas_call(
        _tt_body,
        grid=(NTB,),
        in_specs=[pl.BlockSpec((EMBED, VB), lambda i: (0, i))],
        out_specs=pl.BlockSpec((RB, 4 * EMBED), lambda i: (i, 0)),
        out_shape=jax.ShapeDtypeStruct((NTB * RB, 4 * EMBED), jnp.float32),
    )(emb_t)


def _tc_fc_body(w_ref, x_ref, b_ref, o_ref):
    o_ref[...] = lax.dot_general(
        w_ref[...], x_ref[...],
        (((1,), (1,)), ((), ())),
        preferred_element_type=jnp.float32,
    ) + b_ref[...]


@jax.jit
def _tc_fc(embed_sum, w_fc, b_col):
    bm = 2048
    out_t = pl.pallas_call(
        _tc_fc_body,
        grid=(BATCH // bm,),
        in_specs=[
            pl.BlockSpec((NCLASS, EMBED), lambda i: (0, 0)),
            pl.BlockSpec((bm, EMBED), lambda i: (i, 0)),
            pl.BlockSpec((NCLASS, 1), lambda i: (0, 0)),
        ],
        out_specs=pl.BlockSpec((NCLASS, bm), lambda i: (0, i)),
        out_shape=jax.ShapeDtypeStruct((NCLASS, BATCH), jnp.float32),
    )(w_fc, embed_sum, b_col)
    return out_t.T


def kernel(words, emb_table, W_fc, b_fc):
    w = words.astype(jnp.int32)
    w2 = (w // VB) * VB + (w % RB) * 4 + (w % VB) // RB
    words_grouped = w2.T.reshape(NW, NCH, GPC, GW)
    table128 = _tc_table_transpose(emb_table.T)
    table_rm = table128.reshape(VOCAB2, EMBED)
    embed_sum = _sc_embed_sum(words_grouped, table_rm)
    return _tc_fc(embed_sum, W_fc, b_fc.reshape(NCLASS, 1))


# speedup vs baseline: 4.7041x; 1.5503x over previous
import functools

import jax
import jax.numpy as jnp
import numpy as np
from jax import lax
from jax.experimental import pallas as pl
from jax.experimental.pallas import tpu as pltpu
from jax.experimental.pallas import tpu_sc as plsc

VOCAB = 1000000
NCLASS = 1000
EMBED = 32
CTX = 50
BATCH = 16384

NC = 2
NS = 16
NW = NC * NS
BPW = BATCH // NW
BC = 32
NCH = BPW // BC
GW = 100
GPC = BC * CTX // GW
HALF = EMBED // 2


def _sc_embed_sum_body(words_hbm, emb_hbm, out_hbm, idx_v, rows_v, out_v, sem_g):
    w = lax.axis_index("s") * NC + lax.axis_index("c")

    def chunk_body(ch, _):
        pltpu.sync_copy(words_hbm.at[w, ch], idx_v)
        copies = []
        for g in range(GPC):
            copies.append(pltpu.async_copy(
                emb_hbm.at[idx_v.at[g]],
                rows_v.at[pl.ds(g * GW, GW)],
                sem_g,
            ))
        for cp in copies:
            cp.wait()

        def elem_body(e, _):
            r0 = e * CTX
            a0 = rows_v[r0, pl.ds(0, HALF)]
            a1 = rows_v[r0, pl.ds(HALF, HALF)]
            b0 = rows_v[r0 + 1, pl.ds(0, HALF)]
            b1 = rows_v[r0 + 1, pl.ds(HALF, HALF)]
            for c in range(2, CTX, 2):
                a0 = a0 + rows_v[r0 + c, pl.ds(0, HALF)]
                a1 = a1 + rows_v[r0 + c, pl.ds(HALF, HALF)]
                b0 = b0 + rows_v[r0 + c + 1, pl.ds(0, HALF)]
                b1 = b1 + rows_v[r0 + c + 1, pl.ds(HALF, HALF)]
            out_v[e, pl.ds(0, HALF)] = a0 + b0
            out_v[e, pl.ds(HALF, HALF)] = a1 + b1
            return _

        lax.fori_loop(0, BC, elem_body, 0, unroll=False)
        pltpu.sync_copy(out_v, out_hbm.at[pl.ds(w * BPW + ch * BC, BC)])
        return _

    lax.fori_loop(0, NCH, chunk_body, 0, unroll=False)


@jax.jit
def _sc_embed_sum(words_grouped, emb_table):
    return pl.kernel(
        _sc_embed_sum_body,
        out_type=jax.ShapeDtypeStruct((BATCH, EMBED), jnp.float32),
        mesh=plsc.VectorSubcoreMesh(
            core_axis_name="c", subcore_axis_name="s",
            num_cores=NC, num_subcores=NS),
        scratch_types=[
            pltpu.VMEM((GPC, GW), jnp.int32),
            pltpu.VMEM((BC * CTX, EMBED), jnp.float32),
            pltpu.VMEM((BC, EMBED), jnp.float32),
            pltpu.SemaphoreType.DMA,
        ],
        compiler_params=pltpu.CompilerParams(use_tc_tiling_on_sc=False),
    )(words_grouped, emb_table)


VB = 16384
RB = VB // 4
NTB = (VOCAB + VB - 1) // VB
VOCAB2 = NTB * VB


def _tt_body(x0_ref, x1_ref, x2_ref, x3_ref, o_ref):
    stacked = jnp.concatenate(
        [x0_ref[...], x1_ref[...], x2_ref[...], x3_ref[...]], axis=0)
    o_ref[...] = stacked.T


@jax.jit
def _tc_table_transpose(emb_t):
    last_full = VOCAB // RB - 1

    def slab(j):
        if j == 0:
            return pl.BlockSpec((EMBED, RB), lambda i: (0, 4 * i))
        return pl.BlockSpec(
            (EMBED, RB),
            lambda i, j=j: (0, jnp.minimum(4 * i + j, last_full)))
    return pl.pallas_call(
        _tt_body,
        grid=(NTB,),
        in_specs=[slab(0), slab(1), slab(2), slab(3)],
        out_specs=pl.BlockSpec((RB, 4 * EMBED), lambda i: (i, 0)),
        out_shape=jax.ShapeDtypeStruct((NTB * RB, 4 * EMBED), jnp.float32),
    )(emb_t, emb_t, emb_t, emb_t)


def _tc_fc_body(w_ref, x_ref, b_ref, o_ref):
    o_ref[...] = lax.dot_general(
        w_ref[...], x_ref[...],
        (((1,), (1,)), ((), ())),
        preferred_element_type=jnp.float32,
    ) + b_ref[...]


@jax.jit
def _tc_fc(embed_sum, w_fc, b_col):
    bm = 2048
    out_t = pl.pallas_call(
        _tc_fc_body,
        grid=(BATCH // bm,),
        in_specs=[
            pl.BlockSpec((NCLASS, EMBED), lambda i: (0, 0)),
            pl.BlockSpec((bm, EMBED), lambda i: (i, 0)),
            pl.BlockSpec((NCLASS, 1), lambda i: (0, 0)),
        ],
        out_specs=pl.BlockSpec((NCLASS, bm), lambda i: (0, i)),
        out_shape=jax.ShapeDtypeStruct((NCLASS, BATCH), jnp.float32),
    )(w_fc, embed_sum, b_col)
    return out_t.T


def kernel(words, emb_table, W_fc, b_fc):
    w = words.astype(jnp.int32)
    w2 = (w // VB) * VB + (w % RB) * 4 + (w % VB) // RB
    words_grouped = w2.T.reshape(NW, NCH, GPC, GW)
    table128 = _tc_table_transpose(emb_table.T)
    table_rm = table128.reshape(VOCAB2, EMBED)
    embed_sum = _sc_embed_sum(words_grouped, table_rm)
    return _tc_fc(embed_sum, W_fc, b_fc.reshape(NCLASS, 1))


# speedup vs baseline: 5.3428x; 1.1358x over previous
import functools

import jax
import jax.numpy as jnp
import numpy as np
from jax import lax
from jax.experimental import pallas as pl
from jax.experimental.pallas import tpu as pltpu
from jax.experimental.pallas import tpu_sc as plsc

VOCAB = 1000000
NCLASS = 1000
EMBED = 32
CTX = 50
BATCH = 16384

NC = 2
NS = 16
NW = NC * NS
BPW = BATCH // NW
BC = 32
NCH = BPW // BC
GW = 100
GPC = BC * CTX // GW
HALF = EMBED // 2


def _sc_embed_sum_body(words_hbm, emb_hbm, out_hbm,
                       idx0, idx1, rows0, rows1, out0, out1,
                       sem_g0, sem_g1):
    w = lax.axis_index("s") * NC + lax.axis_index("c")
    idx_v = (idx0, idx1)
    rows_v = (rows0, rows1)
    out_v = (out0, out1)
    sem_g = (sem_g0, sem_g1)

    def fire(ch, buf):
        pltpu.sync_copy(words_hbm.at[w, ch], idx_v[buf])
        return [
            pltpu.async_copy(
                emb_hbm.at[idx_v[buf].at[g]],
                rows_v[buf].at[pl.ds(g * GW, GW)],
                sem_g[buf],
            )
            for g in range(GPC)
        ]

    def accum(ch, buf, copies):
        for cp in copies:
            cp.wait()
        rows = rows_v[buf]
        out = out_v[buf]

        def elem_body(e, _):
            r0 = e * CTX
            a0 = rows[r0, pl.ds(0, HALF)]
            a1 = rows[r0, pl.ds(HALF, HALF)]
            b0 = rows[r0 + 1, pl.ds(0, HALF)]
            b1 = rows[r0 + 1, pl.ds(HALF, HALF)]
            for c in range(2, CTX, 2):
                a0 = a0 + rows[r0 + c, pl.ds(0, HALF)]
                a1 = a1 + rows[r0 + c, pl.ds(HALF, HALF)]
                b0 = b0 + rows[r0 + c + 1, pl.ds(0, HALF)]
                b1 = b1 + rows[r0 + c + 1, pl.ds(HALF, HALF)]
            out[e, pl.ds(0, HALF)] = a0 + b0
            out[e, pl.ds(HALF, HALF)] = a1 + b1
            return _

        lax.fori_loop(0, BC, elem_body, 0, unroll=False)
        pltpu.sync_copy(out, out_hbm.at[pl.ds(w * BPW + ch * BC, BC)])

    copies = {0: fire(0, 0)}
    for ch in range(NCH):
        if ch + 1 < NCH:
            copies[(ch + 1) % 2] = fire(ch + 1, (ch + 1) % 2)
        accum(ch, ch % 2, copies[ch % 2])


@jax.jit
def _sc_embed_sum(words_grouped, emb_table):
    return pl.kernel(
        _sc_embed_sum_body,
        out_type=jax.ShapeDtypeStruct((BATCH, EMBED), jnp.float32),
        mesh=plsc.VectorSubcoreMesh(
            core_axis_name="c", subcore_axis_name="s",
            num_cores=NC, num_subcores=NS),
        scratch_types=[
            pltpu.VMEM((GPC, GW), jnp.int32),
            pltpu.VMEM((GPC, GW), jnp.int32),
            pltpu.VMEM((BC * CTX, EMBED), jnp.float32),
            pltpu.VMEM((BC * CTX, EMBED), jnp.float32),
            pltpu.VMEM((BC, EMBED), jnp.float32),
            pltpu.VMEM((BC, EMBED), jnp.float32),
            pltpu.SemaphoreType.DMA,
            pltpu.SemaphoreType.DMA,
        ],
        compiler_params=pltpu.CompilerParams(use_tc_tiling_on_sc=False),
    )(words_grouped, emb_table)


VB = 16384
RB = VB // 4
NTB = (VOCAB + VB - 1) // VB
VOCAB2 = NTB * VB


def _tt_body(x0_ref, x1_ref, x2_ref, x3_ref, o_ref):
    stacked = jnp.concatenate(
        [x0_ref[...], x1_ref[...], x2_ref[...], x3_ref[...]], axis=0)
    o_ref[...] = stacked.T


@jax.jit
def _tc_table_transpose(emb_t):
    last_full = VOCAB // RB - 1

    def slab(j):
        if j == 0:
            return pl.BlockSpec((EMBED, RB), lambda i: (0, 4 * i))
        return pl.BlockSpec(
            (EMBED, RB),
            lambda i, j=j: (0, jnp.minimum(4 * i + j, last_full)))
    return pl.pallas_call(
        _tt_body,
        grid=(NTB,),
        in_specs=[slab(0), slab(1), slab(2), slab(3)],
        out_specs=pl.BlockSpec((RB, 4 * EMBED), lambda i: (i, 0)),
        out_shape=jax.ShapeDtypeStruct((NTB * RB, 4 * EMBED), jnp.float32),
    )(emb_t, emb_t, emb_t, emb_t)


def _tc_fc_body(w_ref, x_ref, b_ref, o_ref):
    o_ref[...] = lax.dot_general(
        w_ref[...], x_ref[...],
        (((1,), (1,)), ((), ())),
        preferred_element_type=jnp.float32,
    ) + b_ref[...]


@jax.jit
def _tc_fc(embed_sum, w_fc, b_col):
    bm = 2048
    out_t = pl.pallas_call(
        _tc_fc_body,
        grid=(BATCH // bm,),
        in_specs=[
            pl.BlockSpec((NCLASS, EMBED), lambda i: (0, 0)),
            pl.BlockSpec((bm, EMBED), lambda i: (i, 0)),
            pl.BlockSpec((NCLASS, 1), lambda i: (0, 0)),
        ],
        out_specs=pl.BlockSpec((NCLASS, bm), lambda i: (0, i)),
        out_shape=jax.ShapeDtypeStruct((NCLASS, BATCH), jnp.float32),
    )(w_fc, embed_sum, b_col)
    return out_t.T


def kernel(words, emb_table, W_fc, b_fc):
    w = words.astype(jnp.int32)
    w2 = (w // VB) * VB + (w % RB) * 4 + (w % VB) // RB
    words_grouped = w2.T.reshape(NW, NCH, GPC, GW)
    table128 = _tc_table_transpose(emb_table.T)
    table_rm = table128.reshape(VOCAB2, EMBED)
    embed_sum = _sc_embed_sum(words_grouped, table_rm)
    return _tc_fc(embed_sum, W_fc, b_fc.reshape(NCLASS, 1))


# speedup vs baseline: 5.6877x; 1.0645x over previous
import functools

import jax
import jax.numpy as jnp
import numpy as np
from jax import lax
from jax.experimental import pallas as pl
from jax.experimental.pallas import tpu as pltpu
from jax.experimental.pallas import tpu_sc as plsc

VOCAB = 1000000
NCLASS = 1000
EMBED = 32
CTX = 50
BATCH = 16384

NC = 2
NS = 16
NW = NC * NS
BPW = BATCH // NW
BC = 32
NCH = BPW // BC
CPC = BC * CTX
GW = 64
GPC = CPC // GW
HALF = EMBED // 2


def _sc_embed_sum_body(words_hbm, emb_hbm, out_hbm,
                       idx0, idx1, rows0, rows1, out0, out1,
                       sem_g0, sem_g1):
    w = lax.axis_index("s") * NC + lax.axis_index("c")
    idx_v = (idx0, idx1)
    rows_v = (rows0, rows1)
    out_v = (out0, out1)
    sem_g = (sem_g0, sem_g1)

    def fire(ch, buf):
        pltpu.sync_copy(words_hbm.at[w, ch], idx_v[buf])
        return [
            pltpu.async_copy(
                emb_hbm.at[idx_v[buf].at[g]],
                rows_v[buf].at[pl.ds(g * GW, GW)],
                sem_g[buf],
            )
            for g in range(GPC)
        ]

    def accum(ch, buf, copies):
        for cp in copies:
            cp.wait()
        rows = rows_v[buf]
        out = out_v[buf]

        def elem_body(e, _):
            a0 = rows[e, pl.ds(0, HALF)]
            a1 = rows[e, pl.ds(HALF, HALF)]
            b0 = rows[BC + e, pl.ds(0, HALF)]
            b1 = rows[BC + e, pl.ds(HALF, HALF)]
            for c in range(2, CTX, 2):
                a0 = a0 + rows[c * BC + e, pl.ds(0, HALF)]
                a1 = a1 + rows[c * BC + e, pl.ds(HALF, HALF)]
                b0 = b0 + rows[(c + 1) * BC + e, pl.ds(0, HALF)]
                b1 = b1 + rows[(c + 1) * BC + e, pl.ds(HALF, HALF)]
            out[e, pl.ds(0, HALF)] = a0 + b0
            out[e, pl.ds(HALF, HALF)] = a1 + b1
            return _

        lax.fori_loop(0, BC, elem_body, 0, unroll=False)
        pltpu.sync_copy(out, out_hbm.at[pl.ds(w * BPW + ch * BC, BC)])

    copies = {0: fire(0, 0)}
    for ch in range(NCH):
        if ch + 1 < NCH:
            copies[(ch + 1) % 2] = fire(ch + 1, (ch + 1) % 2)
        accum(ch, ch % 2, copies[ch % 2])


def _sc_idx_prep_body(words_hbm, out_hbm, ws0, ws1, ob, sem0, sem1):
    w = lax.axis_index("s") * NC + lax.axis_index("c")
    ws = (ws0, ws1)
    sems = (sem0, sem1)

    def fire(ch, buf):
        pltpu.async_copy(
            words_hbm.at[:, pl.ds(w * BPW + ch * BC, BC)], ws[buf], sems[buf])

    def wait(buf):
        pltpu.make_async_copy(
            words_hbm.at[:, pl.ds(0, BC)], ws[buf], sems[buf]).wait()

    def compute(ch, buf):
        for c in range(CTX):
            for k in range(BC // 16):
                v = ws[buf][c, pl.ds(16 * k, 16)]
                v2 = ((v & ~(VB - 1)) + ((v & (RB - 1)) << 2)
                      + ((v & (VB - 1)) >> 12))
                ob[pl.ds(c * BC + 16 * k, 16)] = v2
        pltpu.sync_copy(ob, out_hbm.at[w, pl.ds(ch * CPC, CPC)])

    fire(0, 0)

    def pair_body(p, _):
        ch0 = 2 * p
        fire(ch0 + 1, 1)
        wait(0)
        compute(ch0, 0)
        fire(jnp.minimum(ch0 + 2, NCH - 1), 0)
        wait(1)
        compute(ch0 + 1, 1)
        return _

    lax.fori_loop(0, NCH // 2, pair_body, 0, unroll=False)
    wait(0)


@jax.jit
def _sc_idx_prep(words):
    return pl.kernel(
        _sc_idx_prep_body,
        out_type=jax.ShapeDtypeStruct((NW, BPW * CTX), jnp.int32),
        mesh=plsc.VectorSubcoreMesh(
            core_axis_name="c", subcore_axis_name="s",
            num_cores=NC, num_subcores=NS),
        scratch_types=[
            pltpu.VMEM((CTX, BC), jnp.int32),
            pltpu.VMEM((CTX, BC), jnp.int32),
            pltpu.VMEM((CPC,), jnp.int32),
            pltpu.SemaphoreType.DMA,
            pltpu.SemaphoreType.DMA,
        ],
        compiler_params=pltpu.CompilerParams(use_tc_tiling_on_sc=False),
    )(words)


@jax.jit
def _sc_embed_sum(words_grouped, emb_table):
    return pl.kernel(
        _sc_embed_sum_body,
        out_type=jax.ShapeDtypeStruct((BATCH, EMBED), jnp.float32),
        mesh=plsc.VectorSubcoreMesh(
            core_axis_name="c", subcore_axis_name="s",
            num_cores=NC, num_subcores=NS),
        scratch_types=[
            pltpu.VMEM((GPC, GW), jnp.int32),
            pltpu.VMEM((GPC, GW), jnp.int32),
            pltpu.VMEM((BC * CTX, EMBED), jnp.float32),
            pltpu.VMEM((BC * CTX, EMBED), jnp.float32),
            pltpu.VMEM((BC, EMBED), jnp.float32),
            pltpu.VMEM((BC, EMBED), jnp.float32),
            pltpu.SemaphoreType.DMA,
            pltpu.SemaphoreType.DMA,
        ],
        compiler_params=pltpu.CompilerParams(use_tc_tiling_on_sc=False),
    )(words_grouped, emb_table)


VB = 16384
RB = VB // 4
NTB = (VOCAB + VB - 1) // VB
VOCAB2 = NTB * VB


def _tt_body(x0_ref, x1_ref, x2_ref, x3_ref, o_ref):
    stacked = jnp.concatenate(
        [x0_ref[...], x1_ref[...], x2_ref[...], x3_ref[...]], axis=0)
    o_ref[...] = stacked.T


@jax.jit
def _tc_table_transpose(emb_t):
    last_full = VOCAB // RB - 1

    def slab(j):
        if j == 0:
            return pl.BlockSpec((EMBED, RB), lambda i: (0, 4 * i))
        return pl.BlockSpec(
            (EMBED, RB),
            lambda i, j=j: (0, jnp.minimum(4 * i + j, last_full)))
    return pl.pallas_call(
        _tt_body,
        grid=(NTB,),
        in_specs=[slab(0), slab(1), slab(2), slab(3)],
        out_specs=pl.BlockSpec((RB, 4 * EMBED), lambda i: (i, 0)),
        out_shape=jax.ShapeDtypeStruct((NTB * RB, 4 * EMBED), jnp.float32),
    )(emb_t, emb_t, emb_t, emb_t)


def _tc_fc_body(w_ref, x_ref, b_ref, o_ref):
    o_ref[...] = lax.dot_general(
        w_ref[...], x_ref[...],
        (((1,), (1,)), ((), ())),
        preferred_element_type=jnp.float32,
    ) + b_ref[...]


@jax.jit
def _tc_fc(embed_sum, w_fc, b_col):
    bm = 2048
    out_t = pl.pallas_call(
        _tc_fc_body,
        grid=(BATCH // bm,),
        in_specs=[
            pl.BlockSpec((NCLASS, EMBED), lambda i: (0, 0)),
            pl.BlockSpec((bm, EMBED), lambda i: (i, 0)),
            pl.BlockSpec((NCLASS, 1), lambda i: (0, 0)),
        ],
        out_specs=pl.BlockSpec((NCLASS, bm), lambda i: (0, i)),
        out_shape=jax.ShapeDtypeStruct((NCLASS, BATCH), jnp.float32),
    )(w_fc, embed_sum, b_col)
    return out_t.T


def kernel(words, emb_table, W_fc, b_fc):
    idx_flat = _sc_idx_prep(words.astype(jnp.int32))
    words_grouped = idx_flat.reshape(NW, NCH, GPC, GW)
    table128 = _tc_table_transpose(emb_table.T)
    table_rm = table128.reshape(VOCAB2, EMBED)
    embed_sum = _sc_embed_sum(words_grouped, table_rm)
    return _tc_fc(embed_sum, W_fc, b_fc.reshape(NCLASS, 1))


# speedup vs baseline: 6.0435x; 1.0625x over previous
import functools

import jax
import jax.numpy as jnp
import numpy as np
from jax import lax
from jax.experimental import pallas as pl
from jax.experimental.pallas import tpu as pltpu
from jax.experimental.pallas import tpu_sc as plsc

VOCAB = 1000000
NCLASS = 1000
EMBED = 32
CTX = 50
BATCH = 16384

NC = 2
NS = 16
NW = NC * NS
BPW = BATCH // NW
BC = 32
NCH = BPW // BC
CPC = BC * CTX
GW = 64
GPC = CPC // GW
HALF = EMBED // 2


def _sc_embed_sum_body(words_hbm, emb_hbm, out_hbm,
                       idx0, idx1, rows0, rows1, out0, out1,
                       sem_g0, sem_g1):
    w = lax.axis_index("s") * NC + lax.axis_index("c")
    idx_v = (idx0, idx1)
    rows_v = (rows0, rows1)
    out_v = (out0, out1)
    sem_g = (sem_g0, sem_g1)

    def fire(ch, buf):
        pltpu.sync_copy(words_hbm.at[w, ch], idx_v[buf])
        return [
            pltpu.async_copy(
                emb_hbm.at[idx_v[buf].at[g]],
                rows_v[buf].at[pl.ds(g * GW, GW)],
                sem_g[buf],
            )
            for g in range(GPC)
        ]

    def accum(ch, buf, copies):
        for cp in copies:
            cp.wait()
        rows = rows_v[buf]
        out = out_v[buf]

        def elem_body(e, _):
            a0 = rows[e, pl.ds(0, HALF)]
            a1 = rows[e, pl.ds(HALF, HALF)]
            b0 = rows[BC + e, pl.ds(0, HALF)]
            b1 = rows[BC + e, pl.ds(HALF, HALF)]
            for c in range(2, CTX, 2):
                a0 = a0 + rows[c * BC + e, pl.ds(0, HALF)]
                a1 = a1 + rows[c * BC + e, pl.ds(HALF, HALF)]
                b0 = b0 + rows[(c + 1) * BC + e, pl.ds(0, HALF)]
                b1 = b1 + rows[(c + 1) * BC + e, pl.ds(HALF, HALF)]
            out[e, pl.ds(0, HALF)] = a0 + b0
            out[e, pl.ds(HALF, HALF)] = a1 + b1
            return _

        lax.fori_loop(0, BC, elem_body, 0, unroll=False)
        pltpu.sync_copy(out, out_hbm.at[pl.ds(w * BPW + ch * BC, BC)])

    copies = {0: fire(0, 0)}
    for ch in range(NCH):
        if ch + 1 < NCH:
            copies[(ch + 1) % 2] = fire(ch + 1, (ch + 1) % 2)
        accum(ch, ch % 2, copies[ch % 2])


def _sc_idx_prep_body(words_hbm, out_hbm, ws0, ws1, ob, sem0, sem1):
    w = lax.axis_index("s") * NC + lax.axis_index("c")
    ws = (ws0, ws1)
    sems = (sem0, sem1)

    def fire(ch, buf):
        pltpu.async_copy(
            words_hbm.at[:, pl.ds(w * BPW + ch * BC, BC)], ws[buf], sems[buf])

    def wait(buf):
        pltpu.make_async_copy(
            words_hbm.at[:, pl.ds(0, BC)], ws[buf], sems[buf]).wait()

    def compute(ch, buf):
        for c in range(CTX):
            for k in range(BC // 16):
                v = ws[buf][c, pl.ds(16 * k, 16)]
                v2 = ((v & ~(VB - 1)) + ((v & (RB - 1)) << 2)
                      + ((v & (VB - 1)) >> (RB.bit_length() - 1)))
                ob[pl.ds(c * BC + 16 * k, 16)] = v2
        pltpu.sync_copy(ob, out_hbm.at[w, pl.ds(ch * CPC, CPC)])

    fire(0, 0)

    def pair_body(p, _):
        ch0 = 2 * p
        fire(ch0 + 1, 1)
        wait(0)
        compute(ch0, 0)
        fire(jnp.minimum(ch0 + 2, NCH - 1), 0)
        wait(1)
        compute(ch0 + 1, 1)
        return _

    lax.fori_loop(0, NCH // 2, pair_body, 0, unroll=False)
    wait(0)


@jax.jit
def _sc_idx_prep(words):
    return pl.kernel(
        _sc_idx_prep_body,
        out_type=jax.ShapeDtypeStruct((NW, BPW * CTX), jnp.int32),
        mesh=plsc.VectorSubcoreMesh(
            core_axis_name="c", subcore_axis_name="s",
            num_cores=NC, num_subcores=NS),
        scratch_types=[
            pltpu.VMEM((CTX, BC), jnp.int32),
            pltpu.VMEM((CTX, BC), jnp.int32),
            pltpu.VMEM((CPC,), jnp.int32),
            pltpu.SemaphoreType.DMA,
            pltpu.SemaphoreType.DMA,
        ],
        compiler_params=pltpu.CompilerParams(use_tc_tiling_on_sc=False),
    )(words)


@jax.jit
def _sc_embed_sum(words_grouped, emb_table):
    return pl.kernel(
        _sc_embed_sum_body,
        out_type=jax.ShapeDtypeStruct((BATCH, EMBED), jnp.float32),
        mesh=plsc.VectorSubcoreMesh(
            core_axis_name="c", subcore_axis_name="s",
            num_cores=NC, num_subcores=NS),
        scratch_types=[
            pltpu.VMEM((GPC, GW), jnp.int32),
            pltpu.VMEM((GPC, GW), jnp.int32),
            pltpu.VMEM((BC * CTX, EMBED), jnp.float32),
            pltpu.VMEM((BC * CTX, EMBED), jnp.float32),
            pltpu.VMEM((BC, EMBED), jnp.float32),
            pltpu.VMEM((BC, EMBED), jnp.float32),
            pltpu.SemaphoreType.DMA,
            pltpu.SemaphoreType.DMA,
        ],
        compiler_params=pltpu.CompilerParams(use_tc_tiling_on_sc=False),
    )(words_grouped, emb_table)


VB = 65536
RB = VB // 4
NTB = (VOCAB + VB - 1) // VB
VOCAB2 = NTB * VB


def _tt_body(x0_ref, x1_ref, x2_ref, x3_ref, o_ref):
    stacked = jnp.concatenate(
        [x0_ref[...], x1_ref[...], x2_ref[...], x3_ref[...]], axis=0)
    o_ref[...] = stacked.T


@jax.jit
def _tc_table_transpose(emb_t):
    last_full = VOCAB // RB - 1

    def slab(j):
        if j == 0:
            return pl.BlockSpec((EMBED, RB), lambda i: (0, 4 * i))
        return pl.BlockSpec(
            (EMBED, RB),
            lambda i, j=j: (0, jnp.minimum(4 * i + j, last_full)))
    return pl.pallas_call(
        _tt_body,
        grid=(NTB,),
        in_specs=[slab(0), slab(1), slab(2), slab(3)],
        out_specs=pl.BlockSpec((RB, 4 * EMBED), lambda i: (i, 0)),
        out_shape=jax.ShapeDtypeStruct((NTB * RB, 4 * EMBED), jnp.float32),
    )(emb_t, emb_t, emb_t, emb_t)


def _tc_fc_body(w_ref, x_ref, b_ref, o_ref):
    o_ref[...] = lax.dot_general(
        w_ref[...], x_ref[...],
        (((1,), (1,)), ((), ())),
        preferred_element_type=jnp.float32,
    ) + b_ref[...]


@jax.jit
def _tc_fc(embed_sum, w_fc, b_col):
    bm = 2048
    out_t = pl.pallas_call(
        _tc_fc_body,
        grid=(BATCH // bm,),
        in_specs=[
            pl.BlockSpec((NCLASS, EMBED), lambda i: (0, 0)),
            pl.BlockSpec((bm, EMBED), lambda i: (i, 0)),
            pl.BlockSpec((NCLASS, 1), lambda i: (0, 0)),
        ],
        out_specs=pl.BlockSpec((NCLASS, bm), lambda i: (0, i)),
        out_shape=jax.ShapeDtypeStruct((NCLASS, BATCH), jnp.float32),
    )(w_fc, embed_sum, b_col)
    return out_t.T


def kernel(words, emb_table, W_fc, b_fc):
    idx_flat = _sc_idx_prep(words.astype(jnp.int32))
    words_grouped = idx_flat.reshape(NW, NCH, GPC, GW)
    table128 = _tc_table_transpose(emb_table.T)
    table_rm = table128.reshape(VOCAB2, EMBED)
    embed_sum = _sc_embed_sum(words_grouped, table_rm)
    return _tc_fc(embed_sum, W_fc, b_fc.reshape(NCLASS, 1))


# speedup vs baseline: 6.1485x; 1.0174x over previous
import functools

import jax
import jax.numpy as jnp
import numpy as np
from jax import lax
from jax.experimental import pallas as pl
from jax.experimental.pallas import tpu as pltpu
from jax.experimental.pallas import tpu_sc as plsc

VOCAB = 1000000
NCLASS = 1000
EMBED = 32
CTX = 50
BATCH = 16384

NC = 2
NS = 16
NW = NC * NS
BPW = BATCH // NW
BC = 32
NCH = BPW // BC
CPC = BC * CTX
GW = 64
GPC = CPC // GW
HALF = EMBED // 2


def _sc_embed_sum_body(words_hbm, emb_hbm, out_hbm,
                       idx0, idx1, rows0, rows1, out0, out1,
                       sem_g0, sem_g1):
    w = lax.axis_index("s") * NC + lax.axis_index("c")
    idx_v = (idx0, idx1)
    rows_v = (rows0, rows1)
    out_v = (out0, out1)
    sem_g = (sem_g0, sem_g1)

    def fire(ch, buf):
        pltpu.sync_copy(words_hbm.at[w, ch], idx_v[buf])
        return [
            pltpu.async_copy(
                emb_hbm.at[idx_v[buf].at[g]],
                rows_v[buf].at[pl.ds(g * GW, GW)],
                sem_g[buf],
            )
            for g in range(GPC)
        ]

    def accum(ch, buf, copies):
        for cp in copies:
            cp.wait()
        rows = rows_v[buf]
        out = out_v[buf]

        def elem_body(e, _):
            a0 = rows[e, pl.ds(0, HALF)]
            a1 = rows[e, pl.ds(HALF, HALF)]
            b0 = rows[BC + e, pl.ds(0, HALF)]
            b1 = rows[BC + e, pl.ds(HALF, HALF)]
            for c in range(2, CTX, 2):
                a0 = a0 + rows[c * BC + e, pl.ds(0, HALF)]
                a1 = a1 + rows[c * BC + e, pl.ds(HALF, HALF)]
                b0 = b0 + rows[(c + 1) * BC + e, pl.ds(0, HALF)]
                b1 = b1 + rows[(c + 1) * BC + e, pl.ds(HALF, HALF)]
            out[e, pl.ds(0, HALF)] = a0 + b0
            out[e, pl.ds(HALF, HALF)] = a1 + b1
            return _

        lax.fori_loop(0, BC, elem_body, 0, unroll=False)
        pltpu.sync_copy(out, out_hbm.at[pl.ds(w * BPW + ch * BC, BC)])

    copies = {0: fire(0, 0)}
    for ch in range(NCH):
        if ch + 1 < NCH:
            copies[(ch + 1) % 2] = fire(ch + 1, (ch + 1) % 2)
        accum(ch, ch % 2, copies[ch % 2])


def _sc_idx_prep_body(words_hbm, out_hbm, ws0, ws1, ob, sem0, sem1):
    w = lax.axis_index("s") * NC + lax.axis_index("c")
    ws = (ws0, ws1)
    sems = (sem0, sem1)

    def fire(ch, buf):
        pltpu.async_copy(
            words_hbm.at[:, pl.ds(w * BPW + ch * BC, BC)], ws[buf], sems[buf])

    def wait(buf):
        pltpu.make_async_copy(
            words_hbm.at[:, pl.ds(0, BC)], ws[buf], sems[buf]).wait()

    def compute(ch, buf):
        for c in range(CTX):
            for k in range(BC // 16):
                v = ws[buf][c, pl.ds(16 * k, 16)]
                v2 = ((v & ~(VB - 1)) + ((v & (RB - 1)) << 2)
                      + ((v & (VB - 1)) >> (RB.bit_length() - 1)))
                ob[pl.ds(c * BC + 16 * k, 16)] = v2
        pltpu.sync_copy(ob, out_hbm.at[w, pl.ds(ch * CPC, CPC)])

    fire(0, 0)

    def pair_body(p, _):
        ch0 = 2 * p
        fire(ch0 + 1, 1)
        wait(0)
        compute(ch0, 0)
        fire(jnp.minimum(ch0 + 2, NCH - 1), 0)
        wait(1)
        compute(ch0 + 1, 1)
        return _

    lax.fori_loop(0, NCH // 2, pair_body, 0, unroll=False)
    wait(0)


@jax.jit
def _sc_idx_prep(words):
    return pl.kernel(
        _sc_idx_prep_body,
        out_type=jax.ShapeDtypeStruct((NW, BPW * CTX), jnp.int32),
        mesh=plsc.VectorSubcoreMesh(
            core_axis_name="c", subcore_axis_name="s",
            num_cores=NC, num_subcores=NS),
        scratch_types=[
            pltpu.VMEM((CTX, BC), jnp.int32),
            pltpu.VMEM((CTX, BC), jnp.int32),
            pltpu.VMEM((CPC,), jnp.int32),
            pltpu.SemaphoreType.DMA,
            pltpu.SemaphoreType.DMA,
        ],
        compiler_params=pltpu.CompilerParams(use_tc_tiling_on_sc=False),
    )(words)


@jax.jit
def _sc_embed_sum(words_grouped, emb_table):
    return pl.kernel(
        _sc_embed_sum_body,
        out_type=jax.ShapeDtypeStruct((BATCH, EMBED), jnp.float32),
        mesh=plsc.VectorSubcoreMesh(
            core_axis_name="c", subcore_axis_name="s",
            num_cores=NC, num_subcores=NS),
        scratch_types=[
            pltpu.VMEM((GPC, GW), jnp.int32),
            pltpu.VMEM((GPC, GW), jnp.int32),
            pltpu.VMEM((BC * CTX, EMBED), jnp.float32),
            pltpu.VMEM((BC * CTX, EMBED), jnp.float32),
            pltpu.VMEM((BC, EMBED), jnp.float32),
            pltpu.VMEM((BC, EMBED), jnp.float32),
            pltpu.SemaphoreType.DMA,
            pltpu.SemaphoreType.DMA,
        ],
        compiler_params=pltpu.CompilerParams(use_tc_tiling_on_sc=False),
    )(words_grouped, emb_table)


VB = 65536
RB = VB // 4
NTB = (VOCAB + VB - 1) // VB
VOCAB2 = NTB * VB


def _tt_body(x0_ref, x1_ref, x2_ref, x3_ref, o_ref):
    stacked = jnp.concatenate(
        [x0_ref[...], x1_ref[...], x2_ref[...], x3_ref[...]], axis=0)
    o_ref[...] = stacked.T


@jax.jit
def _tc_table_transpose(emb_t):
    last_ok = (VOCAB + RB - 1) // RB - 1

    def slab(j):
        if j == 0:
            return pl.BlockSpec((EMBED, RB), lambda i: (0, 4 * i))
        return pl.BlockSpec(
            (EMBED, RB),
            lambda i, j=j: (0, jnp.minimum(4 * i + j, last_ok)))
    return pl.pallas_call(
        _tt_body,
        grid=(NTB,),
        in_specs=[slab(0), slab(1), slab(2), slab(3)],
        out_specs=pl.BlockSpec((RB, 4 * EMBED), lambda i: (i, 0)),
        out_shape=jax.ShapeDtypeStruct((NTB * RB, 4 * EMBED), jnp.float32),
    )(emb_t, emb_t, emb_t, emb_t)


def _tc_fc_body(w_ref, x_ref, b_ref, o_ref):
    o_ref[...] = lax.dot_general(
        w_ref[...], x_ref[...],
        (((1,), (1,)), ((), ())),
        preferred_element_type=jnp.float32,
    ) + b_ref[...]


@jax.jit
def _tc_fc(embed_sum, w_fc, b_col):
    bm = 2048
    out_t = pl.pallas_call(
        _tc_fc_body,
        grid=(BATCH // bm,),
        in_specs=[
            pl.BlockSpec((NCLASS, EMBED), lambda i: (0, 0)),
            pl.BlockSpec((bm, EMBED), lambda i: (i, 0)),
            pl.BlockSpec((NCLASS, 1), lambda i: (0, 0)),
        ],
        out_specs=pl.BlockSpec((NCLASS, bm), lambda i: (0, i)),
        out_shape=jax.ShapeDtypeStruct((NCLASS, BATCH), jnp.float32),
    )(w_fc, embed_sum, b_col)
    return out_t.T


def kernel(words, emb_table, W_fc, b_fc):
    idx_flat = _sc_idx_prep(words.astype(jnp.int32))
    words_grouped = idx_flat.reshape(NW, NCH, GPC, GW)
    table128 = _tc_table_transpose(emb_table.T)
    table_rm = table128.reshape(VOCAB2, EMBED)
    embed_sum = _sc_embed_sum(words_grouped, table_rm)
    return _tc_fc(embed_sum, W_fc, b_fc.reshape(NCLASS, 1))


# speedup vs baseline: 6.3954x; 1.0402x over previous
import functools

import jax
import jax.numpy as jnp
import numpy as np
from jax import lax
from jax.experimental import pallas as pl
from jax.experimental.pallas import tpu as pltpu
from jax.experimental.pallas import tpu_sc as plsc

VOCAB = 1000000
NCLASS = 1000
EMBED = 32
CTX = 50
BATCH = 16384

NC = 2
NS = 16
NW = NC * NS
BPW = BATCH // NW
BC = 32
NCH = BPW // BC
CPC = BC * CTX
GW = 64
GPC = CPC // GW
HALF = EMBED // 2


def _sc_embed_sum_body(words_hbm, emb_hbm, out_hbm,
                       idx0, idx1, rows0, rows1, out0, out1,
                       sem_g0, sem_g1):
    w = lax.axis_index("s") * NC + lax.axis_index("c")
    idx_v = (idx0, idx1)
    rows_v = (rows0, rows1)
    out_v = (out0, out1)
    sem_g = (sem_g0, sem_g1)

    def fire(ch, buf):
        pltpu.sync_copy(words_hbm.at[w, ch], idx_v[buf])
        return [
            pltpu.async_copy(
                emb_hbm.at[idx_v[buf].at[g]],
                rows_v[buf].at[pl.ds(g * GW, GW)],
                sem_g[buf],
            )
            for g in range(GPC)
        ]

    zero = jnp.zeros((HALF,), jnp.float32)

    def zero_body(e, _):
        for h in range(2 * HALF, 8 * HALF, HALF):
            out0[e, pl.ds(h, HALF)] = zero
            out1[e, pl.ds(h, HALF)] = zero
        return _

    lax.fori_loop(0, BC, zero_body, 0, unroll=False)

    def accum(ch, buf, copies):
        for cp in copies:
            cp.wait()
        rows = rows_v[buf]
        out = out_v[buf]

        def elem_body(e, _):
            a0 = rows[e, pl.ds(0, HALF)]
            a1 = rows[e, pl.ds(HALF, HALF)]
            b0 = rows[BC + e, pl.ds(0, HALF)]
            b1 = rows[BC + e, pl.ds(HALF, HALF)]
            for c in range(2, CTX, 2):
                a0 = a0 + rows[c * BC + e, pl.ds(0, HALF)]
                a1 = a1 + rows[c * BC + e, pl.ds(HALF, HALF)]
                b0 = b0 + rows[(c + 1) * BC + e, pl.ds(0, HALF)]
                b1 = b1 + rows[(c + 1) * BC + e, pl.ds(HALF, HALF)]
            out[e, pl.ds(0, HALF)] = a0 + b0
            out[e, pl.ds(HALF, HALF)] = a1 + b1
            return _

        lax.fori_loop(0, BC, elem_body, 0, unroll=False)
        pltpu.sync_copy(out, out_hbm.at[pl.ds(w * BPW + ch * BC, BC)])

    copies = {0: fire(0, 0)}
    for ch in range(NCH):
        if ch + 1 < NCH:
            copies[(ch + 1) % 2] = fire(ch + 1, (ch + 1) % 2)
        accum(ch, ch % 2, copies[ch % 2])


def _sc_idx_prep_body(words_hbm, out_hbm, ws0, ws1, ob, sem0, sem1):
    w = lax.axis_index("s") * NC + lax.axis_index("c")
    ws = (ws0, ws1)
    sems = (sem0, sem1)

    def fire(ch, buf):
        pltpu.async_copy(
            words_hbm.at[:, pl.ds(w * BPW + ch * BC, BC)], ws[buf], sems[buf])

    def wait(buf):
        pltpu.make_async_copy(
            words_hbm.at[:, pl.ds(0, BC)], ws[buf], sems[buf]).wait()

    def compute(ch, buf):
        for c in range(CTX):
            for k in range(BC // 16):
                v = ws[buf][c, pl.ds(16 * k, 16)]
                v2 = ((v & ~(VB - 1)) + ((v & (RB - 1)) << 2)
                      + ((v & (VB - 1)) >> (RB.bit_length() - 1)))
                ob[pl.ds(c * BC + 16 * k, 16)] = v2
        pltpu.sync_copy(ob, out_hbm.at[w, pl.ds(ch * CPC, CPC)])

    fire(0, 0)

    def pair_body(p, _):
        ch0 = 2 * p
        fire(ch0 + 1, 1)
        wait(0)
        compute(ch0, 0)
        fire(jnp.minimum(ch0 + 2, NCH - 1), 0)
        wait(1)
        compute(ch0 + 1, 1)
        return _

    lax.fori_loop(0, NCH // 2, pair_body, 0, unroll=False)
    wait(0)


@jax.jit
def _sc_idx_prep(words):
    return pl.kernel(
        _sc_idx_prep_body,
        out_type=jax.ShapeDtypeStruct((NW, BPW * CTX), jnp.int32),
        mesh=plsc.VectorSubcoreMesh(
            core_axis_name="c", subcore_axis_name="s",
            num_cores=NC, num_subcores=NS),
        scratch_types=[
            pltpu.VMEM((CTX, BC), jnp.int32),
            pltpu.VMEM((CTX, BC), jnp.int32),
            pltpu.VMEM((CPC,), jnp.int32),
            pltpu.SemaphoreType.DMA,
            pltpu.SemaphoreType.DMA,
        ],
        compiler_params=pltpu.CompilerParams(use_tc_tiling_on_sc=False),
    )(words)


@jax.jit
def _sc_embed_sum(words_grouped, emb_table):
    return pl.kernel(
        _sc_embed_sum_body,
        out_type=jax.ShapeDtypeStruct((BATCH, 4 * EMBED), jnp.float32),
        mesh=plsc.VectorSubcoreMesh(
            core_axis_name="c", subcore_axis_name="s",
            num_cores=NC, num_subcores=NS),
        scratch_types=[
            pltpu.VMEM((GPC, GW), jnp.int32),
            pltpu.VMEM((GPC, GW), jnp.int32),
            pltpu.VMEM((BC * CTX, EMBED), jnp.float32),
            pltpu.VMEM((BC * CTX, EMBED), jnp.float32),
            pltpu.VMEM((BC, 4 * EMBED), jnp.float32),
            pltpu.VMEM((BC, 4 * EMBED), jnp.float32),
            pltpu.SemaphoreType.DMA,
            pltpu.SemaphoreType.DMA,
        ],
        compiler_params=pltpu.CompilerParams(use_tc_tiling_on_sc=False),
    )(words_grouped, emb_table)


VB = 65536
RB = VB // 4
NTB = (VOCAB + VB - 1) // VB
VOCAB2 = NTB * VB


def _tt_body(x0_ref, x1_ref, x2_ref, x3_ref, o_ref):
    stacked = jnp.concatenate(
        [x0_ref[...], x1_ref[...], x2_ref[...], x3_ref[...]], axis=0)
    o_ref[...] = stacked.T


@jax.jit
def _tc_table_transpose(emb_t):
    last_ok = (VOCAB + RB - 1) // RB - 1

    def slab(j):
        if j == 0:
            return pl.BlockSpec((EMBED, RB), lambda i: (0, 4 * i))
        return pl.BlockSpec(
            (EMBED, RB),
            lambda i, j=j: (0, jnp.minimum(4 * i + j, last_ok)))
    return pl.pallas_call(
        _tt_body,
        grid=(NTB,),
        in_specs=[slab(0), slab(1), slab(2), slab(3)],
        out_specs=pl.BlockSpec((RB, 4 * EMBED), lambda i: (i, 0)),
        out_shape=jax.ShapeDtypeStruct((NTB * RB, 4 * EMBED), jnp.float32),
    )(emb_t, emb_t, emb_t, emb_t)


def _tc_fc_body(w_ref, x_ref, b_ref, o_ref):
    o_ref[...] = lax.dot_general(
        w_ref[...], x_ref[...],
        (((1,), (1,)), ((), ())),
        preferred_element_type=jnp.float32,
    ) + b_ref[...]


@jax.jit
def _tc_fc(embed_sum, w_fc, b_col):
    bm = 2048
    out_t = pl.pallas_call(
        _tc_fc_body,
        grid=(BATCH // bm,),
        in_specs=[
            pl.BlockSpec((NCLASS, 4 * EMBED), lambda i: (0, 0)),
            pl.BlockSpec((bm, 4 * EMBED), lambda i: (i, 0)),
            pl.BlockSpec((NCLASS, 1), lambda i: (0, 0)),
        ],
        out_specs=pl.BlockSpec((NCLASS, bm), lambda i: (0, i)),
        out_shape=jax.ShapeDtypeStruct((NCLASS, BATCH), jnp.float32),
    )(w_fc, embed_sum, b_col)
    return out_t.T


def kernel(words, emb_table, W_fc, b_fc):
    idx_flat = _sc_idx_prep(words.astype(jnp.int32))
    words_grouped = idx_flat.reshape(NW, NCH, GPC, GW)
    table128 = _tc_table_transpose(emb_table.T)
    table_rm = table128.reshape(VOCAB2, EMBED)
    embed_pad = _sc_embed_sum(words_grouped, table_rm)
    w_pad = jnp.pad(W_fc, ((0, 0), (0, 3 * EMBED)))
    return _tc_fc(embed_pad, w_pad, b_fc.reshape(NCLASS, 1))
